# baseline (speedup 1.0000x reference)
import jax
import jax.numpy as jnp
import numpy as np
from jax import lax
from jax.experimental import pallas as pl
from jax.experimental.pallas import tpu as pltpu
from jax.experimental.pallas import tpu_sc as plsc

N = 10000
NPAD = 10112
NTBL = 10112
K = 64
OUTW = 48
ROWW = 128
NC, NS = 2, 16
NW = NC * NS
NBLK = 79
UPW = NBLK * K // NW
SCALE = 0.1

_F = np.arange(OUTW)
_PAT = np.zeros((8, 128), np.int32)
_PAT[0, :OUTW] = _F % 12
_PAT[1, :OUTW] = 3 * (_F // 12) + _F % 3
_PAT[3, :16] = np.arange(16)


def _body(x_hbm, e_hbm, c_hbm, pat_hbm, out_hbm,
          c_v, idx0, idx1, rows0, rows1, ost0, ost1, xit_v, pat_v,
          sI0, sI1, sG0, sG1, sO0, sO1):
  wid = lax.axis_index("s") * NC + lax.axis_index("c")

  idx_v = [idx0, idx1]
  rows_v = [rows0, rows1]
  ost_v = [ost0, ost1]
  sI = [sI0, sI1]
  sG = [sG0, sG1]
  sO = [sO0, sO1]

  pltpu.sync_copy(c_hbm, c_v)
  pltpu.sync_copy(pat_hbm, pat_v)

  zv = pat_v[2, pl.ds(0, 16)]
  iv = pat_v[3, pl.ds(0, 16)]
  rl = [iv + l * 16 for l in range(8)]

  uS = wid * UPW
  uE = uS + UPW
  gS = uS // K
  gE = (uE - 1) // K

  def compute(b, n0, nl):
    ev = idx_v[b]
    rv = rows_v[b]
    ov = ost_v[b]
    s16 = []
    for l in range(nl):
      e16 = ev[pl.ds(l * 16, 16)]
      cj = plsc.load_gather(c_v, [e16])
      ci = c_v[pl.ds(n0 + l * 16, 16)]
      mi = (ci > 0).astype(jnp.float32) * SCALE
      s16.append(mi * (cj > 0).astype(jnp.float32))

    @plsc.parallel_loop(0, OUTW, 1, unroll=4)
    def _comp(f):
      fs = zv + f
      ca = plsc.load_gather(pat_v, [zv, fs])
      cb = plsc.load_gather(pat_v, [zv + 1, fs])
      for l in range(nl):
        a = plsc.load_gather(rv, [rl[l], ca])
        xi = plsc.load_gather(xit_v, [rl[l], cb])
        ov[f, pl.ds(l * 16, 16)] = (a - xi) * s16[l]

  def start_idx(kc, n0, b):
    pltpu.make_async_copy(e_hbm.at[kc, pl.ds(n0, 128)], idx_v[b],
                          sI[b]).start()

  def wait_idx(b):
    pltpu.make_async_copy(e_hbm.at[0, pl.ds(0, 128)], idx_v[b],
                          sI[b]).wait()

  def start_gather(b):
    pltpu.make_async_copy(x_hbm.at[idx_v[b]], rows_v[b], sG[b]).start()

  def wait_gather(b):
    pltpu.make_async_copy(x_hbm.at[idx_v[b]], rows_v[b], sG[b]).wait()

  def start_out(kc, n0, b):
    pltpu.make_async_copy(ost_v[b], out_hbm.at[0, kc, :, pl.ds(n0, 128)],
                          sO[b]).start()

  def wait_out(b):
    pltpu.make_async_copy(ost_v[b], out_hbm.at[0, 0, :, pl.ds(0, 128)],
                          sO[b]).wait()

  def block_body(g, _):
    n0 = g * 128
    klo = jnp.maximum(uS - g * K, 0)
    khi = jnp.minimum(uE - g * K, K)
    trip = khi - klo
    npair = (trip + 1) // 2

    pltpu.sync_copy(x_hbm.at[pl.ds(n0, 128)], xit_v)

    def kc(t):
      return klo + jnp.minimum(t, trip - 1)

    start_idx(kc(0), n0, 0)
    start_idx(kc(1), n0, 1)
    wait_idx(0)
    start_gather(0)
    pltpu.make_async_copy(out_hbm.at[0, 0, :, pl.ds(0, 128)], ost_v[1],
                          sO[1]).start()

    def pair_body(p, _):
      for b in range(2):
        t = 2 * p + b
        ob = 1 - b
        wait_gather(b)
        wait_out(ob)
        wait_idx(ob)
        start_gather(ob)
        compute(b, n0, 8)
        start_out(kc(t), n0, b)
        start_idx(kc(t + 2), n0, b)
      return 0

    lax.fori_loop(0, npair, pair_body, 0)

    wait_out(1)
    wait_gather(0)
    wait_idx(1)
    return 0

  lax.fori_loop(gS, gE + 1, block_body, 0)


@jax.jit
def _run(x128, eT, c, pat):
  mesh = plsc.VectorSubcoreMesh(core_axis_name="c", subcore_axis_name="s")
  f = pl.kernel(
      _body,
      out_type=jax.ShapeDtypeStruct((1, K, OUTW, NPAD), jnp.float32),
      mesh=mesh,
      compiler_params=pltpu.CompilerParams(needs_layout_passes=False),
      scratch_types=[
          pltpu.VMEM((NPAD,), jnp.int32),
          pltpu.VMEM((128,), jnp.int32),
          pltpu.VMEM((128,), jnp.int32),
          pltpu.VMEM((128, ROWW), jnp.float32),
          pltpu.VMEM((128, ROWW), jnp.float32),
          pltpu.VMEM((OUTW, 128), jnp.float32),
          pltpu.VMEM((OUTW, 128), jnp.float32),
          pltpu.VMEM((128, ROWW), jnp.float32),
          pltpu.VMEM((8, 128), jnp.int32),
          pltpu.SemaphoreType.DMA,
          pltpu.SemaphoreType.DMA,
          pltpu.SemaphoreType.DMA,
          pltpu.SemaphoreType.DMA,
          pltpu.SemaphoreType.DMA,
          pltpu.SemaphoreType.DMA,
      ],
  )
  return f(x128, eT, c, pat)


def kernel(X, edge_idx, C):
  B = X.shape[0]
  x12 = X.reshape(N, 12)
  x48 = jnp.concatenate([x12, x12, x12, x12], axis=1)
  x128 = jnp.pad(x48, ((0, NTBL - N), (0, ROWW - OUTW)))
  eT = jnp.pad(edge_idx.reshape(N, K).astype(jnp.int32).T,
               ((0, 0), (0, NPAD - N)))
  c = jnp.pad(C.reshape(N).astype(jnp.int32), (0, NPAD - N))
  out = _run(x128, eT, c, jnp.asarray(_PAT))
  return jnp.transpose(out, (0, 3, 1, 2))[:, :N]


# speedup vs baseline: 1.7350x; 1.7350x over previous
import jax
import jax.numpy as jnp
import numpy as np
from jax import lax
from jax.experimental import pallas as pl
from jax.experimental.pallas import tpu as pltpu
from jax.experimental.pallas import tpu_sc as plsc

N = 10000
NPAD = 10016
K = 64
OUTW = 48
ROWW = 128
NC, NS = 2, 16
NW = NC * NS
NB = 2
EC = NB * K
NGRP = EC // 128
NCHUNK = N // NB
TT = 158
SCALE = 0.1

_PAT = np.zeros((8, 128), np.int32)
_PAT[0:3, :16] = (3 * (np.arange(OUTW) // 12)
                  + np.arange(OUTW) % 3).reshape(3, 16)
_PAT[4, :16] = np.arange(16)


def _body(x_hbm, e_hbm, c_hbm, pat_hbm, out_hbm,
          c_v, idx0, idx1, xiidx0, xiidx1, xi0, xi1, rows0, rows1,
          ost0, ost1, s_v, pat_v,
          sI0, sI1, sX0, sX1, sG0, sG1, sO0, sO1):
  wid = lax.axis_index("s") * NC + lax.axis_index("c")
  nch = (NCHUNK - wid + NW - 1) // NW

  idx_v = [idx0, idx1]
  xiidx = [xiidx0, xiidx1]
  xi_v = [xi0, xi1]
  rows_v = [rows0, rows1]
  ost_v = [ost0, ost1]
  sI = [sI0, sI1]
  sX = [sX0, sX1]
  sG = [sG0, sG1]
  sO = [sO0, sO1]

  pltpu.sync_copy(c_hbm, c_v)
  pltpu.sync_copy(pat_hbm, pat_v)

  ib = [pat_v[r, pl.ds(0, 16)] for r in range(3)]
  zv = pat_v[3, pl.ds(0, 16)]
  iv = pat_v[4, pl.ds(0, 16)]

  def chunk_of(t):
    return wid + jnp.minimum(t, nch - 1) * NW

  def start_ix(t, b):
    ch = chunk_of(t)
    pltpu.make_async_copy(
        e_hbm.at[pl.ds(ch * EC, EC)], idx_v[b], sI[b]).start()
    xiidx[b][:] = iv + ch * NB
    pltpu.make_async_copy(x_hbm.at[xiidx[b]], xi_v[b], sX[b]).start()

  def wait_ix_sem(b):
    pltpu.make_async_copy(
        e_hbm.at[pl.ds(0, EC)], idx_v[b], sI[b]).wait()
    pltpu.make_async_copy(x_hbm.at[xiidx[b]], xi_v[b], sX[b]).wait()

  def start_gather(b):
    for g in range(NGRP):
      pltpu.make_async_copy(
          x_hbm.at[idx_v[b].at[pl.ds(g * 128, 128)]],
          rows_v[b].at[pl.ds(g * 128, 128)], sG[b]).start()

  def wait_gather(b):
    for g in range(NGRP):
      pltpu.make_async_copy(
          x_hbm.at[idx_v[b].at[pl.ds(g * 128, 128)]],
          rows_v[b].at[pl.ds(g * 128, 128)], sG[b]).wait()

  def start_out(t, b):
    n0 = chunk_of(t) * NB
    pltpu.make_async_copy(ost_v[b], out_hbm.at[0, pl.ds(n0, NB)],
                          sO[b]).start()

  def wait_out(b):
    pltpu.make_async_copy(ost_v[b], out_hbm.at[0, pl.ds(0, NB)],
                          sO[b]).wait()

  def compute(t, b):
    rv = rows_v[b]
    ov = ost_v[b]
    xv = xi_v[b]
    ev = idx_v[b]
    n0 = chunk_of(t) * NB
    for i in range(NB):
      n = n0 + i
      bvecs = [plsc.load_gather(xv, [zv + i, ib[r]]) for r in range(3)]
      mi = plsc.load_gather(c_v, [zv + n])
      smi = (mi > 0).astype(jnp.float32) * SCALE
      for g in range(K // 16):
        e16 = ev[pl.ds(i * K + g * 16, 16)]
        cj = plsc.load_gather(c_v, [e16])
        s_v[pl.ds(g * 16, 16)] = smi * (cj > 0).astype(jnp.float32)

      @plsc.parallel_loop(0, K, 1, unroll=8)
      def _edge(e):
        row = i * K + e
        sv = plsc.load_gather(s_v, [zv + e])
        for r in range(3):
          a = rv[row, pl.ds(r * 16, 16)]
          ov[i, e, pl.ds(r * 16, 16)] = (a - bvecs[r]) * sv

  start_ix(0, 0)
  start_ix(1, 1)
  wait_ix_sem(0)
  start_gather(0)
  pltpu.make_async_copy(out_hbm.at[0, pl.ds(0, NB)], ost_v[1], sO[1]).start()

  def pair_body(p, _):
    for b in range(2):
      t = 2 * p + b
      ob = 1 - b
      wait_gather(b)
      wait_out(ob)
      wait_ix_sem(ob)
      start_gather(ob)
      compute(t, b)
      start_out(t, b)
      start_ix(t + 2, b)
    return 0

  lax.fori_loop(0, TT // 2, pair_body, 0)

  wait_out(1)
  wait_gather(0)
  wait_ix_sem(1)


@jax.jit
def _run(x128, eflat, c, pat):
  mesh = plsc.VectorSubcoreMesh(core_axis_name="c", subcore_axis_name="s")
  f = pl.kernel(
      _body,
      out_type=jax.ShapeDtypeStruct((1, N, K, OUTW), jnp.float32),
      mesh=mesh,
      compiler_params=pltpu.CompilerParams(needs_layout_passes=False),
      scratch_types=[
          pltpu.VMEM((N,), jnp.int32),
          pltpu.VMEM((EC,), jnp.int32),
          pltpu.VMEM((EC,), jnp.int32),
          pltpu.VMEM((16,), jnp.int32),
          pltpu.VMEM((16,), jnp.int32),
          pltpu.VMEM((16, ROWW), jnp.float32),
          pltpu.VMEM((16, ROWW), jnp.float32),
          pltpu.VMEM((EC, ROWW), jnp.float32),
          pltpu.VMEM((EC, ROWW), jnp.float32),
          pltpu.VMEM((NB, K, OUTW), jnp.float32),
          pltpu.VMEM((NB, K, OUTW), jnp.float32),
          pltpu.VMEM((K,), jnp.float32),
          pltpu.VMEM((8, 128), jnp.int32),
          pltpu.SemaphoreType.DMA,
          pltpu.SemaphoreType.DMA,
          pltpu.SemaphoreType.DMA,
          pltpu.SemaphoreType.DMA,
          pltpu.SemaphoreType.DMA,
          pltpu.SemaphoreType.DMA,
          pltpu.SemaphoreType.DMA,
          pltpu.SemaphoreType.DMA,
      ],
  )
  return f(x128, eflat, c, pat)


def kernel(X, edge_idx, C):
  B = X.shape[0]
  x12 = X.reshape(N, 12)
  x48 = jnp.concatenate([x12, x12, x12, x12], axis=1)
  x128 = jnp.pad(x48, ((0, NPAD - N), (0, ROWW - OUTW)))
  eflat = edge_idx.reshape(N * K).astype(jnp.int32)
  c = C.reshape(N).astype(jnp.int32)
  return _run(x128, eflat, c, jnp.asarray(_PAT))
